# baseline (speedup 1.0000x reference)
import functools

import jax
import jax.numpy as jnp
from jax.experimental import pallas as pl
from jax.experimental.pallas import tpu as pltpu


def _pick_bm(n: int, target: int = 512) -> int:
    bm = 8
    for cand in range(8, target + 1, 8):
        if n % cand == 0:
            bm = cand
    return bm


def _support0_kernel(x_ref, w_ref, s_ref):
    xb = x_ref[...].astype(jnp.bfloat16)
    s_ref[...] = jnp.dot(
        xb, w_ref[...], preferred_element_type=jnp.float32
    ).astype(jnp.bfloat16)


def _layer_kernel(adj_ref, s_ref, b_ref, wn_ref, ml_ref, sn_ref, z_ref):
    a = adj_ref[...].astype(jnp.bfloat16)
    out = jnp.dot(a, s_ref[...], preferred_element_type=jnp.float32)
    h = jnp.maximum(out + b_ref[...], 0.0)
    hb = h.astype(jnp.bfloat16)
    sn_ref[...] = jnp.dot(
        hb, wn_ref[...], preferred_element_type=jnp.float32
    ).astype(jnp.bfloat16)
    z_ref[...] = jnp.dot(hb, ml_ref[...], preferred_element_type=jnp.float32)


def _last_layer_kernel(adj_ref, s_ref, b_ref, ml_ref, z_ref):
    a = adj_ref[...].astype(jnp.bfloat16)
    out = jnp.dot(a, s_ref[...], preferred_element_type=jnp.float32)
    h = jnp.maximum(out + b_ref[...], 0.0)
    hb = h.astype(jnp.bfloat16)
    z_ref[...] = jnp.dot(hb, ml_ref[...], preferred_element_type=jnp.float32)


def _final_kernel(z1_ref, z2_ref, z3_ref, bl_ref, o_ref):
    z = z1_ref[...] + z2_ref[...] + z3_ref[...] + bl_ref[...]
    m = jnp.max(z, axis=1, keepdims=True)
    zs = z - m
    o_ref[...] = zs - jnp.log(jnp.sum(jnp.exp(zs), axis=1, keepdims=True))


def kernel(x, adj, W0, b0, W1, b1, W2, b2, Wl, bl):
    n, f = x.shape
    nclass = Wl.shape[0]
    bm = _pick_bm(n)
    grid = n // bm

    w0b = W0.astype(jnp.bfloat16)
    w1b = W1.astype(jnp.bfloat16)
    w2b = W2.astype(jnp.bfloat16)
    wlt = Wl.T
    m1 = wlt[0 * f:1 * f].astype(jnp.bfloat16)
    m2 = wlt[1 * f:2 * f].astype(jnp.bfloat16)
    m3 = wlt[2 * f:3 * f].astype(jnp.bfloat16)
    b0r = b0.reshape(1, f)
    b1r = b1.reshape(1, f)
    b2r = b2.reshape(1, f)
    blr = bl.reshape(1, nclass)

    bm0 = _pick_bm(n, 2048)
    s0 = pl.pallas_call(
        _support0_kernel,
        grid=(n // bm0,),
        in_specs=[
            pl.BlockSpec((bm0, f), lambda i: (i, 0)),
            pl.BlockSpec((f, f), lambda i: (0, 0)),
        ],
        out_specs=pl.BlockSpec((bm0, f), lambda i: (i, 0)),
        out_shape=jax.ShapeDtypeStruct((n, f), jnp.bfloat16),
    )(x, w0b)

    def mid_layer(s, b, wn, ml):
        return pl.pallas_call(
            _layer_kernel,
            grid=(grid,),
            in_specs=[
                pl.BlockSpec((bm, n), lambda i: (i, 0)),
                pl.BlockSpec((n, f), lambda i: (0, 0)),
                pl.BlockSpec((1, f), lambda i: (0, 0)),
                pl.BlockSpec((f, f), lambda i: (0, 0)),
                pl.BlockSpec((f, nclass), lambda i: (0, 0)),
            ],
            out_specs=(
                pl.BlockSpec((bm, f), lambda i: (i, 0)),
                pl.BlockSpec((bm, nclass), lambda i: (i, 0)),
            ),
            out_shape=(
                jax.ShapeDtypeStruct((n, f), jnp.bfloat16),
                jax.ShapeDtypeStruct((n, nclass), jnp.float32),
            ),
        )(adj, s, b, wn, ml)

    s1, z1 = mid_layer(s0, b0r, w1b, m1)
    s2, z2 = mid_layer(s1, b1r, w2b, m2)

    z3 = pl.pallas_call(
        _last_layer_kernel,
        grid=(grid,),
        in_specs=[
            pl.BlockSpec((bm, n), lambda i: (i, 0)),
            pl.BlockSpec((n, f), lambda i: (0, 0)),
            pl.BlockSpec((1, f), lambda i: (0, 0)),
            pl.BlockSpec((f, nclass), lambda i: (0, 0)),
        ],
        out_specs=pl.BlockSpec((bm, nclass), lambda i: (i, 0)),
        out_shape=jax.ShapeDtypeStruct((n, nclass), jnp.float32),
    )(adj, s2, b2r, m3)

    bmf = _pick_bm(n, 2048)
    out = pl.pallas_call(
        _final_kernel,
        grid=(n // bmf,),
        in_specs=[
            pl.BlockSpec((bmf, nclass), lambda i: (i, 0)),
            pl.BlockSpec((bmf, nclass), lambda i: (i, 0)),
            pl.BlockSpec((bmf, nclass), lambda i: (i, 0)),
            pl.BlockSpec((1, nclass), lambda i: (0, 0)),
        ],
        out_specs=pl.BlockSpec((bmf, nclass), lambda i: (i, 0)),
        out_shape=jax.ShapeDtypeStruct((n, nclass), jnp.float32),
    )(z1, z2, z3, blr)
    return out


# speedup vs baseline: 1.0702x; 1.0702x over previous
import functools

import jax
import jax.numpy as jnp
from jax.experimental import pallas as pl
from jax.experimental.pallas import tpu as pltpu


def _pick_bm(n: int, target: int = 512) -> int:
    bm = 8
    for cand in range(8, target + 1, 8):
        if n % cand == 0:
            bm = cand
    return bm


def _support0_kernel(x_ref, w_ref, s_ref):
    xb = x_ref[...].astype(jnp.bfloat16)
    s_ref[...] = jnp.dot(
        xb, w_ref[...], preferred_element_type=jnp.float32
    ).astype(jnp.bfloat16)


def _epilogue(a, s_ref, b_ref, ml_ref, z_ref, wn_ref=None, sn_ref=None):
    out = jnp.dot(a, s_ref[...], preferred_element_type=jnp.float32)
    h = jnp.maximum(out + b_ref[...], 0.0)
    hb = h.astype(jnp.bfloat16)
    if sn_ref is not None:
        sn_ref[...] = jnp.dot(
            hb, wn_ref[...], preferred_element_type=jnp.float32
        ).astype(jnp.bfloat16)
    z_ref[...] = jnp.dot(hb, ml_ref[...], preferred_element_type=jnp.float32)


def _layer1_kernel(adj_ref, s_ref, b_ref, wn_ref, ml_ref, ab_ref, sn_ref, z_ref):
    a = adj_ref[...].astype(jnp.bfloat16)
    ab_ref[...] = a
    _epilogue(a, s_ref, b_ref, ml_ref, z_ref, wn_ref, sn_ref)


def _layer2_kernel(adjb_ref, s_ref, b_ref, wn_ref, ml_ref, sn_ref, z_ref):
    _epilogue(adjb_ref[...], s_ref, b_ref, ml_ref, z_ref, wn_ref, sn_ref)


def _layer3_kernel(adjb_ref, s_ref, b_ref, ml_ref, z_ref):
    _epilogue(adjb_ref[...], s_ref, b_ref, ml_ref, z_ref)


def _final_kernel(z1_ref, z2_ref, z3_ref, bl_ref, o_ref):
    z = z1_ref[...] + z2_ref[...] + z3_ref[...] + bl_ref[...]
    m = jnp.max(z, axis=1, keepdims=True)
    zs = z - m
    o_ref[...] = zs - jnp.log(jnp.sum(jnp.exp(zs), axis=1, keepdims=True))


def kernel(x, adj, W0, b0, W1, b1, W2, b2, Wl, bl):
    n, f = x.shape
    nclass = Wl.shape[0]

    w0b = W0.astype(jnp.bfloat16)
    w1b = W1.astype(jnp.bfloat16)
    w2b = W2.astype(jnp.bfloat16)
    wlt = Wl.T
    m1 = wlt[0 * f:1 * f].astype(jnp.bfloat16)
    m2 = wlt[1 * f:2 * f].astype(jnp.bfloat16)
    m3 = wlt[2 * f:3 * f].astype(jnp.bfloat16)
    b0r = b0.reshape(1, f)
    b1r = b1.reshape(1, f)
    b2r = b2.reshape(1, f)
    blr = bl.reshape(1, nclass)

    bm0 = _pick_bm(n, 2048)
    s0 = pl.pallas_call(
        _support0_kernel,
        grid=(n // bm0,),
        in_specs=[
            pl.BlockSpec((bm0, f), lambda i: (i, 0)),
            pl.BlockSpec((f, f), lambda i: (0, 0)),
        ],
        out_specs=pl.BlockSpec((bm0, f), lambda i: (i, 0)),
        out_shape=jax.ShapeDtypeStruct((n, f), jnp.bfloat16),
    )(x, w0b)

    bm1 = _pick_bm(n, 256)
    adjb, s1, z1 = pl.pallas_call(
        _layer1_kernel,
        grid=(n // bm1,),
        in_specs=[
            pl.BlockSpec((bm1, n), lambda i: (i, 0)),
            pl.BlockSpec((n, f), lambda i: (0, 0)),
            pl.BlockSpec((1, f), lambda i: (0, 0)),
            pl.BlockSpec((f, f), lambda i: (0, 0)),
            pl.BlockSpec((f, nclass), lambda i: (0, 0)),
        ],
        out_specs=(
            pl.BlockSpec((bm1, n), lambda i: (i, 0)),
            pl.BlockSpec((bm1, f), lambda i: (i, 0)),
            pl.BlockSpec((bm1, nclass), lambda i: (i, 0)),
        ),
        out_shape=(
            jax.ShapeDtypeStruct((n, n), jnp.bfloat16),
            jax.ShapeDtypeStruct((n, f), jnp.bfloat16),
            jax.ShapeDtypeStruct((n, nclass), jnp.float32),
        ),
    )(adj, s0, b0r, w1b, m1)

    bm = _pick_bm(n, 512)
    s2, z2 = pl.pallas_call(
        _layer2_kernel,
        grid=(n // bm,),
        in_specs=[
            pl.BlockSpec((bm, n), lambda i: (i, 0)),
            pl.BlockSpec((n, f), lambda i: (0, 0)),
            pl.BlockSpec((1, f), lambda i: (0, 0)),
            pl.BlockSpec((f, f), lambda i: (0, 0)),
            pl.BlockSpec((f, nclass), lambda i: (0, 0)),
        ],
        out_specs=(
            pl.BlockSpec((bm, f), lambda i: (i, 0)),
            pl.BlockSpec((bm, nclass), lambda i: (i, 0)),
        ),
        out_shape=(
            jax.ShapeDtypeStruct((n, f), jnp.bfloat16),
            jax.ShapeDtypeStruct((n, nclass), jnp.float32),
        ),
    )(adjb, s1, b1r, w2b, m2)

    z3 = pl.pallas_call(
        _layer3_kernel,
        grid=(n // bm,),
        in_specs=[
            pl.BlockSpec((bm, n), lambda i: (i, 0)),
            pl.BlockSpec((n, f), lambda i: (0, 0)),
            pl.BlockSpec((1, f), lambda i: (0, 0)),
            pl.BlockSpec((f, nclass), lambda i: (0, 0)),
        ],
        out_specs=pl.BlockSpec((bm, nclass), lambda i: (i, 0)),
        out_shape=jax.ShapeDtypeStruct((n, nclass), jnp.float32),
    )(adjb, s2, b2r, m3)

    bmf = _pick_bm(n, 2048)
    out = pl.pallas_call(
        _final_kernel,
        grid=(n // bmf,),
        in_specs=[
            pl.BlockSpec((bmf, nclass), lambda i: (i, 0)),
            pl.BlockSpec((bmf, nclass), lambda i: (i, 0)),
            pl.BlockSpec((bmf, nclass), lambda i: (i, 0)),
            pl.BlockSpec((1, nclass), lambda i: (0, 0)),
        ],
        out_specs=pl.BlockSpec((bmf, nclass), lambda i: (i, 0)),
        out_shape=jax.ShapeDtypeStruct((n, nclass), jnp.float32),
    )(z1, z2, z3, blr)
    return out


# speedup vs baseline: 1.1215x; 1.0480x over previous
import functools

import jax
import jax.numpy as jnp
from jax.experimental import pallas as pl
from jax.experimental.pallas import tpu as pltpu


def _pick_bm(n: int, target: int = 512) -> int:
    bm = 8
    for cand in range(8, target + 1, 8):
        if n % cand == 0:
            bm = cand
    return bm


def _support0_kernel(x_ref, w_ref, s_ref):
    xb = x_ref[...].astype(jnp.bfloat16)
    s_ref[...] = jnp.dot(
        xb, w_ref[...], preferred_element_type=jnp.float32
    ).astype(jnp.bfloat16)


def _epilogue(a, s_ref, b_ref, ml_ref, z_ref, wn_ref=None, sn_ref=None):
    out = jnp.dot(a, s_ref[...], preferred_element_type=jnp.float32)
    h = jnp.maximum(out + b_ref[...], 0.0)
    hb = h.astype(jnp.bfloat16)
    if sn_ref is not None:
        sn_ref[...] = jnp.dot(
            hb, wn_ref[...], preferred_element_type=jnp.float32
        ).astype(jnp.bfloat16)
    z_ref[...] = jnp.dot(hb, ml_ref[...], preferred_element_type=jnp.float32)


def _layer1_kernel(adj_ref, s_ref, b_ref, wn_ref, ml_ref, ab_ref, sn_ref, z_ref):
    a = adj_ref[...].astype(jnp.bfloat16)
    ab_ref[...] = a
    _epilogue(a, s_ref, b_ref, ml_ref, z_ref, wn_ref, sn_ref)


def _layer2_kernel(adjb_ref, s_ref, b_ref, wn_ref, ml_ref, sn_ref, z_ref):
    _epilogue(adjb_ref[...], s_ref, b_ref, ml_ref, z_ref, wn_ref, sn_ref)


def _layer3_kernel(adjb_ref, s_ref, b_ref, ml_ref, z_ref):
    _epilogue(adjb_ref[...], s_ref, b_ref, ml_ref, z_ref)


def _final_kernel(z1_ref, z2_ref, z3_ref, bl_ref, o_ref):
    z = z1_ref[...] + z2_ref[...] + z3_ref[...] + bl_ref[...]
    m = jnp.max(z, axis=1, keepdims=True)
    zs = z - m
    o_ref[...] = zs - jnp.log(jnp.sum(jnp.exp(zs), axis=1, keepdims=True))


def kernel(x, adj, W0, b0, W1, b1, W2, b2, Wl, bl):
    n, f = x.shape
    nclass = Wl.shape[0]

    w0b = W0.astype(jnp.bfloat16)
    w1b = W1.astype(jnp.bfloat16)
    w2b = W2.astype(jnp.bfloat16)
    wlt = Wl.T
    m1 = wlt[0 * f:1 * f].astype(jnp.bfloat16)
    m2 = wlt[1 * f:2 * f].astype(jnp.bfloat16)
    m3 = wlt[2 * f:3 * f].astype(jnp.bfloat16)
    b0r = b0.reshape(1, f)
    b1r = b1.reshape(1, f)
    b2r = b2.reshape(1, f)
    blr = bl.reshape(1, nclass)

    bm0 = _pick_bm(n, 2048)
    s0 = pl.pallas_call(
        _support0_kernel,
        grid=(n // bm0,),
        in_specs=[
            pl.BlockSpec((bm0, f), lambda i: (i, 0)),
            pl.BlockSpec((f, f), lambda i: (0, 0)),
        ],
        out_specs=pl.BlockSpec((bm0, f), lambda i: (i, 0)),
        out_shape=jax.ShapeDtypeStruct((n, f), jnp.bfloat16),
    )(x, w0b)

    bm1 = _pick_bm(n, 400)
    adjb, s1, z1 = pl.pallas_call(
        _layer1_kernel,
        grid=(n // bm1,),
        in_specs=[
            pl.BlockSpec((bm1, n), lambda i: (i, 0)),
            pl.BlockSpec((n, f), lambda i: (0, 0)),
            pl.BlockSpec((1, f), lambda i: (0, 0)),
            pl.BlockSpec((f, f), lambda i: (0, 0)),
            pl.BlockSpec((f, nclass), lambda i: (0, 0)),
        ],
        out_specs=(
            pl.BlockSpec((bm1, n), lambda i: (i, 0)),
            pl.BlockSpec((bm1, f), lambda i: (i, 0)),
            pl.BlockSpec((bm1, nclass), lambda i: (i, 0)),
        ),
        out_shape=(
            jax.ShapeDtypeStruct((n, n), jnp.bfloat16),
            jax.ShapeDtypeStruct((n, f), jnp.bfloat16),
            jax.ShapeDtypeStruct((n, nclass), jnp.float32),
        ),
    )(adj, s0, b0r, w1b, m1)

    bm = _pick_bm(n, 1000)
    s2, z2 = pl.pallas_call(
        _layer2_kernel,
        grid=(n // bm,),
        in_specs=[
            pl.BlockSpec((bm, n), lambda i: (i, 0)),
            pl.BlockSpec((n, f), lambda i: (0, 0)),
            pl.BlockSpec((1, f), lambda i: (0, 0)),
            pl.BlockSpec((f, f), lambda i: (0, 0)),
            pl.BlockSpec((f, nclass), lambda i: (0, 0)),
        ],
        out_specs=(
            pl.BlockSpec((bm, f), lambda i: (i, 0)),
            pl.BlockSpec((bm, nclass), lambda i: (i, 0)),
        ),
        out_shape=(
            jax.ShapeDtypeStruct((n, f), jnp.bfloat16),
            jax.ShapeDtypeStruct((n, nclass), jnp.float32),
        ),
    )(adjb, s1, b1r, w2b, m2)

    z3 = pl.pallas_call(
        _layer3_kernel,
        grid=(n // bm,),
        in_specs=[
            pl.BlockSpec((bm, n), lambda i: (i, 0)),
            pl.BlockSpec((n, f), lambda i: (0, 0)),
            pl.BlockSpec((1, f), lambda i: (0, 0)),
            pl.BlockSpec((f, nclass), lambda i: (0, 0)),
        ],
        out_specs=pl.BlockSpec((bm, nclass), lambda i: (i, 0)),
        out_shape=jax.ShapeDtypeStruct((n, nclass), jnp.float32),
    )(adjb, s2, b2r, m3)

    bmf = _pick_bm(n, 2048)
    out = pl.pallas_call(
        _final_kernel,
        grid=(n // bmf,),
        in_specs=[
            pl.BlockSpec((bmf, nclass), lambda i: (i, 0)),
            pl.BlockSpec((bmf, nclass), lambda i: (i, 0)),
            pl.BlockSpec((bmf, nclass), lambda i: (i, 0)),
            pl.BlockSpec((1, nclass), lambda i: (0, 0)),
        ],
        out_specs=pl.BlockSpec((bmf, nclass), lambda i: (i, 0)),
        out_shape=jax.ShapeDtypeStruct((n, nclass), jnp.float32),
    )(z1, z2, z3, blr)
    return out
